# baseline (speedup 1.0000x reference)
import functools

import jax
import jax.numpy as jnp
from jax import lax
from jax.experimental import pallas as pl
from jax.experimental.pallas import tpu as pltpu
from jax.experimental.pallas import tpu_sc as plsc

L = 16
NC = 2
NS = 16
R = 2
HALO = 0


def _jitter_body(B, D, T, H, x_hbm, off_hbm, out_hbm, *refs):
    c = lax.axis_index("c")
    s = lax.axis_index("s")
    w = s * NC + c
    b = w // (NC * NS // B)
    half = w % (NC * NS // B)
    rows = D // (NC * NS // B)
    d0 = half * rows

    idxv = refs[0]
    xb = refs[1:3]
    ob = refs[3:5]
    isems = refs[5:7]
    osems = refs[7:9]
    ngroup = rows // R

    for h in range(T // H):
        seg = h * H
        start = 0 if h == 0 else seg - HALO

        def in_cp(g, slot):
            return pltpu.make_async_copy(
                x_hbm.at[b, pl.ds(d0 + g * R, R), pl.ds(start, H + HALO)],
                xb[slot], isems[slot])

        def out_cp(g, slot):
            return pltpu.make_async_copy(
                ob[slot], out_hbm.at[b, pl.ds(d0 + g * R, R), pl.ds(seg, H)],
                osems[slot])

        in_cp(0, 0).start()
        pltpu.sync_copy(off_hbm.at[b, pl.ds(seg, H)], idxv)

        @plsc.parallel_loop(0, H // L, unroll=4)
        def mk_idx(i):
            base = i * L
            off = idxv[pl.ds(base, L)]
            gidx = lax.iota(jnp.int32, L) + (seg + base - 1) + off
            gidx = jnp.minimum(jnp.maximum(gidx, 0), T - 1)
            idxv[pl.ds(base, L)] = gidx - start

        def outer(i, carry):
            for k in range(2):
                g = i * 2 + k
                slot = k
                nxt = 1 - k

                @pl.when(g + 1 < ngroup)
                def _():
                    in_cp(g + 1, nxt).start()

                in_cp(g, slot).wait()

                @pl.when(g >= 2)
                def _():
                    out_cp(g - 2, slot).wait()

                @plsc.parallel_loop(0, H // L, unroll=16)
                def chunk(j):
                    base = j * L
                    tv = idxv[pl.ds(base, L)]
                    for r in range(R):
                        rv = jnp.full((L,), r, jnp.int32)
                        ob[slot][r, pl.ds(base, L)] = plsc.load_gather(
                            xb[slot], [rv, tv])

                out_cp(g, slot).start()
            return carry

        lax.fori_loop(0, ngroup // 2, outer, 0)
        out_cp(ngroup - 2, 0).wait()
        out_cp(ngroup - 1, 1).wait()


def kernel(x, offsets):
    B, D, T = x.shape
    H = T
    mesh = plsc.VectorSubcoreMesh(core_axis_name="c", subcore_axis_name="s",
                                   num_cores=NC, num_subcores=NS)
    f = pl.kernel(
        functools.partial(_jitter_body, B, D, T, H),
        out_type=jax.ShapeDtypeStruct(x.shape, x.dtype),
        mesh=mesh,
        compiler_params=pltpu.CompilerParams(needs_layout_passes=False),
        scratch_types=(
            [pltpu.VMEM((H,), jnp.int32)] +
            [pltpu.VMEM((R, H + HALO), jnp.float32)] * 2 +
            [pltpu.VMEM((R, H), jnp.float32)] * 2 +
            [pltpu.SemaphoreType.DMA] * 4
        ),
    )
    return f(x, offsets)


# speedup vs baseline: 1.0032x; 1.0032x over previous
import functools

import jax
import jax.numpy as jnp
from jax import lax
from jax.experimental import pallas as pl
from jax.experimental.pallas import tpu as pltpu
from jax.experimental.pallas import tpu_sc as plsc

L = 16
NC = 2
NS = 16
R = 2
HALO = 0


def _jitter_body(B, D, T, H, x_hbm, off_hbm, out_hbm, *refs):
    c = lax.axis_index("c")
    s = lax.axis_index("s")
    w = s * NC + c
    b = w // (NC * NS // B)
    half = w % (NC * NS // B)
    rows = D // (NC * NS // B)
    d0 = half * rows

    idxv = refs[0]
    xb = refs[1:3]
    ob = refs[3:5]
    isems = refs[5:7]
    osems = refs[7:9]
    ngroup = rows // R

    for h in range(T // H):
        seg = h * H
        start = 0 if h == 0 else seg - HALO

        def in_cp(g, slot):
            return pltpu.make_async_copy(
                x_hbm.at[b, pl.ds(d0 + g * R, R), pl.ds(start, H + HALO)],
                xb[slot], isems[slot])

        def out_cp(g, slot):
            return pltpu.make_async_copy(
                ob[slot], out_hbm.at[b, pl.ds(d0 + g * R, R), pl.ds(seg, H)],
                osems[slot])

        in_cp(0, 0).start()
        pltpu.sync_copy(off_hbm.at[b, pl.ds(seg, H)], idxv)

        @plsc.parallel_loop(0, H // L, unroll=4)
        def mk_idx(i):
            base = i * L
            off = idxv[pl.ds(base, L)]
            gidx = lax.iota(jnp.int32, L) + (seg + base - 1) + off
            gidx = jnp.minimum(jnp.maximum(gidx, 0), T - 1)
            idxv[pl.ds(base, L)] = gidx - start

        def outer(i, carry):
            for k in range(2):
                g = i * 2 + k
                slot = k
                nxt = 1 - k

                @pl.when(g + 1 < ngroup)
                def _():
                    in_cp(g + 1, nxt).start()

                in_cp(g, slot).wait()

                @pl.when(g >= 2)
                def _():
                    out_cp(g - 2, slot).wait()

                @plsc.parallel_loop(0, H // L, unroll=8)
                def chunk(j):
                    base = j * L
                    tv = idxv[pl.ds(base, L)]
                    for r in range(R):
                        rv = jnp.full((L,), r, jnp.int32)
                        ob[slot][r, pl.ds(base, L)] = plsc.load_gather(
                            xb[slot], [rv, tv])

                out_cp(g, slot).start()
            return carry

        lax.fori_loop(0, ngroup // 2, outer, 0)
        out_cp(ngroup - 2, 0).wait()
        out_cp(ngroup - 1, 1).wait()


def kernel(x, offsets):
    B, D, T = x.shape
    H = T
    mesh = plsc.VectorSubcoreMesh(core_axis_name="c", subcore_axis_name="s",
                                   num_cores=NC, num_subcores=NS)
    f = pl.kernel(
        functools.partial(_jitter_body, B, D, T, H),
        out_type=jax.ShapeDtypeStruct(x.shape, x.dtype),
        mesh=mesh,
        compiler_params=pltpu.CompilerParams(needs_layout_passes=False),
        scratch_types=(
            [pltpu.VMEM((H,), jnp.int32)] +
            [pltpu.VMEM((R, H + HALO), jnp.float32)] * 2 +
            [pltpu.VMEM((R, H), jnp.float32)] * 2 +
            [pltpu.SemaphoreType.DMA] * 4
        ),
    )
    return f(x, offsets)


# speedup vs baseline: 1.0315x; 1.0282x over previous
import functools

import jax
import jax.numpy as jnp
from jax import lax
from jax.experimental import pallas as pl
from jax.experimental.pallas import tpu as pltpu
from jax.experimental.pallas import tpu_sc as plsc

L = 16
NC = 2
NS = 16
R = 2
HALO = 0


def _jitter_body(B, D, T, H, x_hbm, off_hbm, out_hbm, *refs):
    c = lax.axis_index("c")
    s = lax.axis_index("s")
    w = s * NC + c
    b = w // (NC * NS // B)
    half = w % (NC * NS // B)
    rows = D // (NC * NS // B)
    d0 = half * rows

    idxv = refs[0]
    xb = refs[1:5]
    ob = refs[5:7]
    isems = refs[7:11]
    osems = refs[11:13]
    ngroup = rows // R

    for h in range(T // H):
        seg = h * H
        start = 0 if h == 0 else seg - HALO

        def in_cp(g, slot):
            return pltpu.make_async_copy(
                x_hbm.at[b, pl.ds(d0 + g * R, R), pl.ds(start, H + HALO)],
                xb[slot], isems[slot])

        def out_cp(g, slot):
            return pltpu.make_async_copy(
                ob[slot], out_hbm.at[b, pl.ds(d0 + g * R, R), pl.ds(seg, H)],
                osems[slot])

        in_cp(0, 0).start()
        in_cp(1, 1).start()
        pltpu.sync_copy(off_hbm.at[b, pl.ds(seg, H)], idxv)

        @plsc.parallel_loop(0, H // L, unroll=4)
        def mk_idx(i):
            base = i * L
            off = idxv[pl.ds(base, L)]
            gidx = lax.iota(jnp.int32, L) + (seg + base - 1) + off
            gidx = jnp.minimum(jnp.maximum(gidx, 0), T - 1)
            idxv[pl.ds(base, L)] = gidx - start

        def outer(i, carry):
            for k in range(4):
                g = i * 4 + k
                islot = k
                oslot = k % 2

                @pl.when(g + 2 < ngroup)
                def _():
                    in_cp(g + 2, (k + 2) % 4).start()

                in_cp(g, islot).wait()

                @pl.when(g >= 2)
                def _():
                    out_cp(g - 2, oslot).wait()

                @plsc.parallel_loop(0, H // L, unroll=8)
                def chunk(j):
                    base = j * L
                    tv = idxv[pl.ds(base, L)]
                    for r in range(R):
                        rv = jnp.full((L,), r, jnp.int32)
                        ob[oslot][r, pl.ds(base, L)] = plsc.load_gather(
                            xb[islot], [rv, tv])

                out_cp(g, oslot).start()
            return carry

        lax.fori_loop(0, ngroup // 4, outer, 0)
        out_cp(ngroup - 2, 0).wait()
        out_cp(ngroup - 1, 1).wait()


def kernel(x, offsets):
    B, D, T = x.shape
    H = T
    mesh = plsc.VectorSubcoreMesh(core_axis_name="c", subcore_axis_name="s",
                                   num_cores=NC, num_subcores=NS)
    f = pl.kernel(
        functools.partial(_jitter_body, B, D, T, H),
        out_type=jax.ShapeDtypeStruct(x.shape, x.dtype),
        mesh=mesh,
        compiler_params=pltpu.CompilerParams(needs_layout_passes=False),
        scratch_types=(
            [pltpu.VMEM((H,), jnp.int32)] +
            [pltpu.VMEM((R, H + HALO), jnp.float32)] * 4 +
            [pltpu.VMEM((R, H), jnp.float32)] * 2 +
            [pltpu.SemaphoreType.DMA] * 6
        ),
    )
    return f(x, offsets)


# speedup vs baseline: 1.0349x; 1.0033x over previous
import functools

import jax
import jax.numpy as jnp
from jax import lax
from jax.experimental import pallas as pl
from jax.experimental.pallas import tpu as pltpu
from jax.experimental.pallas import tpu_sc as plsc

L = 16
NC = 2
NS = 16
R = 2
HALO = 0


def _jitter_body(B, D, T, H, x_hbm, off_hbm, out_hbm, *refs):
    c = lax.axis_index("c")
    s = lax.axis_index("s")
    w = s * NC + c
    b = w // (NC * NS // B)
    half = w % (NC * NS // B)
    rows = D // (NC * NS // B)
    d0 = half * rows

    idxv = refs[0]
    xb = refs[1:5]
    ob = refs[5:7]
    isems = refs[7:11]
    osems = refs[11:13]
    ngroup = rows // R

    for h in range(T // H):
        seg = h * H
        start = 0 if h == 0 else seg - HALO

        def in_cp(g, slot):
            return pltpu.make_async_copy(
                x_hbm.at[b, pl.ds(d0 + g * R, R), pl.ds(start, H + HALO)],
                xb[slot], isems[slot])

        def out_cp(g, slot):
            return pltpu.make_async_copy(
                ob[slot], out_hbm.at[b, pl.ds(d0 + g * R, R), pl.ds(seg, H)],
                osems[slot])

        in_cp(0, 0).start()
        in_cp(1, 1).start()
        in_cp(2, 2).start()
        pltpu.sync_copy(off_hbm.at[b, pl.ds(seg, H)], idxv)

        @plsc.parallel_loop(0, H // L, unroll=4)
        def mk_idx(i):
            base = i * L
            off = idxv[pl.ds(base, L)]
            gidx = lax.iota(jnp.int32, L) + (seg + base - 1) + off
            gidx = jnp.minimum(jnp.maximum(gidx, 0), T - 1)
            idxv[pl.ds(base, L)] = gidx - start

        def outer(i, carry):
            for k in range(4):
                g = i * 4 + k
                islot = k
                oslot = k % 2

                @pl.when(g + 3 < ngroup)
                def _():
                    in_cp(g + 3, (k + 3) % 4).start()

                in_cp(g, islot).wait()

                @pl.when(g >= 2)
                def _():
                    out_cp(g - 2, oslot).wait()

                @plsc.parallel_loop(0, H // L, unroll=8)
                def chunk(j):
                    base = j * L
                    tv = idxv[pl.ds(base, L)]
                    for r in range(R):
                        rv = jnp.full((L,), r, jnp.int32)
                        ob[oslot][r, pl.ds(base, L)] = plsc.load_gather(
                            xb[islot], [rv, tv])

                out_cp(g, oslot).start()
            return carry

        lax.fori_loop(0, ngroup // 4, outer, 0)
        out_cp(ngroup - 2, 0).wait()
        out_cp(ngroup - 1, 1).wait()


def kernel(x, offsets):
    B, D, T = x.shape
    H = T
    mesh = plsc.VectorSubcoreMesh(core_axis_name="c", subcore_axis_name="s",
                                   num_cores=NC, num_subcores=NS)
    f = pl.kernel(
        functools.partial(_jitter_body, B, D, T, H),
        out_type=jax.ShapeDtypeStruct(x.shape, x.dtype),
        mesh=mesh,
        compiler_params=pltpu.CompilerParams(needs_layout_passes=False),
        scratch_types=(
            [pltpu.VMEM((H,), jnp.int32)] +
            [pltpu.VMEM((R, H + HALO), jnp.float32)] * 4 +
            [pltpu.VMEM((R, H), jnp.float32)] * 2 +
            [pltpu.SemaphoreType.DMA] * 6
        ),
    )
    return f(x, offsets)
